# Initial kernel scaffold; baseline (speedup 1.0000x reference)
import jax
import jax.numpy as jnp
from jax.experimental import pallas as pl


def kernel(mixture, codebook, prior1, prior2, L1, L2):
    raise NotImplementedError("write your pallas kernel here")


# speedup vs baseline: 1088.2708x; 1088.2708x over previous
import functools

import jax
import jax.numpy as jnp
from jax import lax
from jax.experimental import pallas as pl
from jax.experimental.pallas import tpu as pltpu

K = 1024
D = 64
T = 256
B = 4
NS = 8
NEG = float("-inf")
BIGI = 2**31 - 1
KK = K * K


def _encode_body(mix_ref, cb_ref, codes_ref):
    m = mix_ref[...]
    c = cb_ref[...]
    prod = lax.dot_general(m, c, (((1,), (1,)), ((), ())),
                           preferred_element_type=jnp.float32)
    d = (jnp.sum(m * m, axis=1, keepdims=True) - 2.0 * prod
         + jnp.sum(c * c, axis=1)[None, :])
    dmin = jnp.min(d, axis=1, keepdims=True)
    iota = lax.broadcasted_iota(jnp.int32, d.shape, 1)
    idx = jnp.min(jnp.where(d == dmin, iota, K), axis=1, keepdims=True)
    codes_ref[...] = idx


def _topn(a, raw1, raw2):
    iota = lax.broadcasted_iota(jnp.int32, a.shape, 1)
    js, v1s, v2s = [], [], []
    for _ in range(NS):
        mx = jnp.max(a, axis=1, keepdims=True)
        j = jnp.min(jnp.where(a == mx, iota, K), axis=1, keepdims=True)
        sel = iota == j
        v1s.append(jnp.sum(jnp.where(sel, raw1, 0.0), axis=1, keepdims=True))
        v2s.append(jnp.sum(jnp.where(sel, raw2, 0.0), axis=1, keepdims=True))
        js.append(j)
        a = jnp.where(sel, NEG, a)
    return (jnp.concatenate(js, 1), jnp.concatenate(v1s, 1),
            jnp.concatenate(v2s, 1))


def _merge4(vj, fidx):
    outv, outf = [], []
    for _ in range(4):
        mx = jnp.max(vj)
        f = jnp.min(jnp.where(vj == mx, fidx, BIGI))
        vj = jnp.where(fidx == f, NEG, vj)
        outv.append(mx)
        outf.append(f)
    return outv, outf


def _combos(p1v, p2v, l1v, l2v, ai, ci, scv):
    vs, eis = [], []
    for r in range(NS):
        p1c = p1v[:, r:r + 1]
        l1c = l1v[:, r:r + 1]
        if scv is None:
            v = ((p1c + p2v) + l1c) + l2v
        else:
            v = (((scv + p1c) + p2v) + l1c) + l2v
        vs.append(v)
        eis.append(ai[:, r:r + 1] * K + ci)
    return jnp.concatenate(vs, 1), jnp.concatenate(eis, 1)


def _bs_body(codes_ref, cb_ref, p1_hbm, p2_hbm, l1_hbm, l2_hbm, out_ref,
             p1r_ref, p2r_ref, l1r_ref, l2r_ref,
             h1_ref, h2_ref, bp_ref, z10_ref, z20_ref, sem):
    m0 = codes_ref[0, 0]

    def fetch(table, row, dst, slot, sem_i):
        cp = pltpu.make_async_copy(table.at[pl.ds(row, 1)],
                                   dst.at[pl.ds(slot, 1)], sem.at[sem_i])
        cp.start()
        return cp

    cps = [fetch(p1_hbm, 0, p1r_ref, 0, 0), fetch(p2_hbm, 0, p2r_ref, 0, 4),
           fetch(l1_hbm, m0, l1r_ref, 0, 8), fetch(l2_hbm, m0, l2r_ref, 0, 9)]
    for cp in cps:
        cp.wait()
    p1r = p1r_ref[pl.ds(0, 1), :]
    p2r = p2r_ref[pl.ds(0, 1), :]
    l1r = l1r_ref[...]
    l2r = l2r_ref[...]
    ai, p1v, l1v = _topn(p1r + l1r, p1r, l1r)
    ci, p2v, l2v = _topn(p2r + l2r, p2r, l2r)
    v, ei = _combos(p1v, p2v, l1v, l2v, ai, ci, None)
    sc_l, f_l = _merge4(v, ei)
    carry = []
    for k in range(4):
        z1k = lax.shift_right_logical(f_l[k], 10)
        z2k = jnp.bitwise_and(f_l[k], K - 1)
        z10_ref[k] = z1k
        z20_ref[k] = z2k
        carry += [z1k, z2k, sc_l[k]]

    def step(t, carry):
        (z10, z20, s0, z11, z21, s1, z12, z22, s2, z13, z23, s3) = carry
        m_t = codes_ref[t, 0]
        cps = [fetch(p1_hbm, z10, p1r_ref, 0, 0), fetch(p1_hbm, z11, p1r_ref, 1, 1),
               fetch(p1_hbm, z12, p1r_ref, 2, 2), fetch(p1_hbm, z13, p1r_ref, 3, 3),
               fetch(p2_hbm, z20, p2r_ref, 0, 4), fetch(p2_hbm, z21, p2r_ref, 1, 5),
               fetch(p2_hbm, z22, p2r_ref, 2, 6), fetch(p2_hbm, z23, p2r_ref, 3, 7),
               fetch(l1_hbm, m_t, l1r_ref, 0, 8), fetch(l2_hbm, m_t, l2r_ref, 0, 9)]
        for cp in cps:
            cp.wait()
        P1r = p1r_ref[...]
        P2r = p2r_ref[...]
        L1r = l1r_ref[...]
        L2r = l2r_ref[...]
        ai, p1v, l1v = _topn(P1r + L1r, P1r, jnp.broadcast_to(L1r, (B, K)))
        ci, p2v, l2v = _topn(P2r + L2r, P2r, jnp.broadcast_to(L2r, (B, K)))
        bio = lax.broadcasted_iota(jnp.int32, (B, 1), 0)
        scv = (jnp.where(bio == 0, s0, 0.0) + jnp.where(bio == 1, s1, 0.0)
               + jnp.where(bio == 2, s2, 0.0) + jnp.where(bio == 3, s3, 0.0))
        v, ei = _combos(p1v, p2v, l1v, l2v, ai, ci, scv)
        bio2 = lax.broadcasted_iota(jnp.int32, (B, NS * NS), 0)
        fi = bio2 * KK + ei
        ts_l, f_l = _merge4(v, fi)
        ncarry = []
        for k in range(4):
            fk = f_l[k]
            bk = lax.shift_right_logical(fk, 20)
            nz1 = jnp.bitwise_and(lax.shift_right_logical(fk, 10), K - 1)
            nz2 = jnp.bitwise_and(fk, K - 1)
            h1_ref[t - 1, k] = nz1
            h2_ref[t - 1, k] = nz2
            bp_ref[t - 1, k] = bk
            ncarry += [nz1, nz2, ts_l[k]]
        return tuple(ncarry)

    lax.fori_loop(1, T, step, tuple(carry))

    def back(j, b):
        i = T - 2 - j
        s1 = h1_ref[i, b]
        s2 = h2_ref[i, b]
        out_ref[0, pl.ds(i + 1, 1), :] = cb_ref[pl.ds(s1, 1), :]
        out_ref[1, pl.ds(i + 1, 1), :] = cb_ref[pl.ds(s2, 1), :]
        return bp_ref[i, b]

    b0 = lax.fori_loop(0, T - 1, back, jnp.int32(0))
    out_ref[0, pl.ds(0, 1), :] = cb_ref[pl.ds(z10_ref[b0], 1), :]
    out_ref[1, pl.ds(0, 1), :] = cb_ref[pl.ds(z20_ref[b0], 1), :]


@jax.jit
def kernel(mixture, codebook, prior1, prior2, L1, L2):
    codes = pl.pallas_call(
        _encode_body,
        out_shape=jax.ShapeDtypeStruct((T, 1), jnp.int32),
    )(mixture, codebook)

    out = pl.pallas_call(
        _bs_body,
        in_specs=[
            pl.BlockSpec(memory_space=pltpu.SMEM),
            pl.BlockSpec(memory_space=pltpu.VMEM),
            pl.BlockSpec(memory_space=pl.ANY),
            pl.BlockSpec(memory_space=pl.ANY),
            pl.BlockSpec(memory_space=pl.ANY),
            pl.BlockSpec(memory_space=pl.ANY),
        ],
        out_shape=jax.ShapeDtypeStruct((2, T, D), jnp.float32),
        scratch_shapes=[
            pltpu.VMEM((B, K), jnp.float32),
            pltpu.VMEM((B, K), jnp.float32),
            pltpu.VMEM((1, K), jnp.float32),
            pltpu.VMEM((1, K), jnp.float32),
            pltpu.SMEM((T - 1, B), jnp.int32),
            pltpu.SMEM((T - 1, B), jnp.int32),
            pltpu.SMEM((T - 1, B), jnp.int32),
            pltpu.SMEM((B,), jnp.int32),
            pltpu.SMEM((B,), jnp.int32),
            pltpu.SemaphoreType.DMA((10,)),
        ],
    )(codes, codebook, prior1, prior2, L1, L2)
    return out


# speedup vs baseline: 1668.4285x; 1.5331x over previous
import jax
import jax.numpy as jnp
from jax import lax
from jax.experimental import pallas as pl
from jax.experimental.pallas import tpu as pltpu

K = 1024
D = 64
T = 256
B = 4
NS = 8
NEG = float("-inf")
BIGI = 2**31 - 1
KK = K * K
IMIN = -2**31


def _encode_body(mix_ref, cb_ref, codes_ref):
    m = mix_ref[...]
    c = cb_ref[...]
    prod = lax.dot_general(m, c, (((1,), (1,)), ((), ())),
                           preferred_element_type=jnp.float32)
    d = (jnp.sum(m * m, axis=1, keepdims=True) - 2.0 * prod
         + jnp.sum(c * c, axis=1)[None, :])
    dmin = jnp.min(d, axis=1, keepdims=True)
    iota = lax.broadcasted_iota(jnp.int32, d.shape, 1)
    idx = jnp.min(jnp.where(d == dmin, iota, K), axis=1, keepdims=True)
    codes_ref[...] = idx


def _packed_keys(s):
    b = lax.bitcast_convert_type(s, jnp.int32)
    ks = b ^ (lax.shift_right_arithmetic(b, 31) & 0x7FFFFFFF)
    iota = lax.broadcasted_iota(jnp.int32, s.shape, 1)
    return (ks & ~1023) | (1023 - iota)


def _screen(P, L):
    keys = _packed_keys(P + L)
    jis, pvs, lvs = [], [], []
    for _ in range(NS):
        mx = jnp.max(keys, axis=1, keepdims=True)
        sel = keys == mx
        pvs.append(jnp.sum(jnp.where(sel, P, 0.0), axis=1, keepdims=True))
        lvs.append(jnp.sum(jnp.where(sel, L, 0.0), axis=1, keepdims=True))
        jis.append(1023 - (mx & 1023))
        keys = jnp.where(sel, IMIN, keys)
    return (jnp.concatenate(jis, 1), jnp.concatenate(pvs, 1),
            jnp.concatenate(lvs, 1))


def _merge4(vj, fidx):
    outv, outf = [], []
    for _ in range(4):
        mx = jnp.max(vj)
        f = jnp.min(jnp.where(vj == mx, fidx, BIGI))
        vj = jnp.where(fidx == f, NEG, vj)
        outv.append(mx)
        outf.append(f)
    return outv, outf


def _combos(p1v, p2v, l1v, l2v, ai, ci, scv):
    vs, eis = [], []
    for r in range(NS):
        p1c = p1v[:, r:r + 1]
        l1c = l1v[:, r:r + 1]
        if scv is None:
            v = ((p1c + p2v) + l1c) + l2v
        else:
            v = (((scv + p1c) + p2v) + l1c) + l2v
        vs.append(v)
        eis.append(ai[:, r:r + 1] * K + ci)
    return jnp.concatenate(vs, 1), jnp.concatenate(eis, 1)


def _bs_body(codes_ref, cb_ref, p1_ref, p2_ref, l1_ref, l2_ref, out_ref,
             h1_ref, h2_ref, bp_ref, z10_ref, z20_ref):
    m0 = codes_ref[0, 0]

    Pi = jnp.concatenate([p1_ref[pl.ds(0, 1), :], p2_ref[pl.ds(0, 1), :]], 0)
    Li = jnp.concatenate([l1_ref[pl.ds(m0, 1), :], l2_ref[pl.ds(m0, 1), :]], 0)
    ji, pv, lv = _screen(Pi, Li)
    v, ei = _combos(pv[0:1], pv[1:2], lv[0:1], lv[1:2],
                    ji[0:1], ji[1:2], None)
    sc_l, f_l = _merge4(v, ei)
    carry = []
    for k in range(4):
        z1k = lax.shift_right_logical(f_l[k], 10)
        z2k = jnp.bitwise_and(f_l[k], K - 1)
        z10_ref[k] = z1k
        z20_ref[k] = z2k
        carry += [z1k, z2k, sc_l[k]]

    def step(t, carry):
        (z10, z20, s0, z11, z21, s1, z12, z22, s2, z13, z23, s3) = carry
        m_t = codes_ref[t, 0]
        P = jnp.concatenate(
            [p1_ref[pl.ds(z10, 1), :], p1_ref[pl.ds(z11, 1), :],
             p1_ref[pl.ds(z12, 1), :], p1_ref[pl.ds(z13, 1), :],
             p2_ref[pl.ds(z20, 1), :], p2_ref[pl.ds(z21, 1), :],
             p2_ref[pl.ds(z22, 1), :], p2_ref[pl.ds(z23, 1), :]], 0)
        l1r = l1_ref[pl.ds(m_t, 1), :]
        l2r = l2_ref[pl.ds(m_t, 1), :]
        L = jnp.concatenate([jnp.broadcast_to(l1r, (B, K)),
                             jnp.broadcast_to(l2r, (B, K))], 0)
        ji, pv, lv = _screen(P, L)
        bio = lax.broadcasted_iota(jnp.int32, (B, 1), 0)
        scv = (jnp.where(bio == 0, s0, 0.0) + jnp.where(bio == 1, s1, 0.0)
               + jnp.where(bio == 2, s2, 0.0) + jnp.where(bio == 3, s3, 0.0))
        v, ei = _combos(pv[0:B], pv[B:2 * B], lv[0:B], lv[B:2 * B],
                        ji[0:B], ji[B:2 * B], scv)
        bio2 = lax.broadcasted_iota(jnp.int32, (B, NS * NS), 0)
        fi = bio2 * KK + ei
        ts_l, f_l = _merge4(v, fi)
        ncarry = []
        for k in range(4):
            fk = f_l[k]
            bk = lax.shift_right_logical(fk, 20)
            nz1 = jnp.bitwise_and(lax.shift_right_logical(fk, 10), K - 1)
            nz2 = jnp.bitwise_and(fk, K - 1)
            h1_ref[t - 1, k] = nz1
            h2_ref[t - 1, k] = nz2
            bp_ref[t - 1, k] = bk
            ncarry += [nz1, nz2, ts_l[k]]
        return tuple(ncarry)

    lax.fori_loop(1, T, step, tuple(carry))

    def back(j, b):
        i = T - 2 - j
        s1 = h1_ref[i, b]
        s2 = h2_ref[i, b]
        out_ref[0, pl.ds(i + 1, 1), :] = cb_ref[pl.ds(s1, 1), :]
        out_ref[1, pl.ds(i + 1, 1), :] = cb_ref[pl.ds(s2, 1), :]
        return bp_ref[i, b]

    b0 = lax.fori_loop(0, T - 1, back, jnp.int32(0))
    out_ref[0, pl.ds(0, 1), :] = cb_ref[pl.ds(z10_ref[b0], 1), :]
    out_ref[1, pl.ds(0, 1), :] = cb_ref[pl.ds(z20_ref[b0], 1), :]


@jax.jit
def kernel(mixture, codebook, prior1, prior2, L1, L2):
    codes = pl.pallas_call(
        _encode_body,
        out_shape=jax.ShapeDtypeStruct((T, 1), jnp.int32),
    )(mixture, codebook)

    out = pl.pallas_call(
        _bs_body,
        in_specs=[
            pl.BlockSpec(memory_space=pltpu.SMEM),
            pl.BlockSpec(memory_space=pltpu.VMEM),
            pl.BlockSpec(memory_space=pltpu.VMEM),
            pl.BlockSpec(memory_space=pltpu.VMEM),
            pl.BlockSpec(memory_space=pltpu.VMEM),
            pl.BlockSpec(memory_space=pltpu.VMEM),
        ],
        out_shape=jax.ShapeDtypeStruct((2, T, D), jnp.float32),
        scratch_shapes=[
            pltpu.SMEM((T - 1, B), jnp.int32),
            pltpu.SMEM((T - 1, B), jnp.int32),
            pltpu.SMEM((T - 1, B), jnp.int32),
            pltpu.SMEM((B,), jnp.int32),
            pltpu.SMEM((B,), jnp.int32),
        ],
    )(codes, codebook, prior1, prior2, L1, L2)
    return out


# speedup vs baseline: 2202.5016x; 1.3201x over previous
import jax
import jax.numpy as jnp
from jax import lax
from jax.experimental import pallas as pl
from jax.experimental.pallas import tpu as pltpu
from jax.experimental.pallas import tpu_sc as plsc

import functools

NC_SC = 2
NSUB = 16
NW = NC_SC * NSUB

K = 1024
D = 64
T = 256
B = 4
NS = 8
NEG = float("-inf")
BIGI = 2**31 - 1
KK = K * K
IMIN = -2**31


def _encode_body(mix_ref, cb_ref, codes_ref):
    m = mix_ref[...]
    c = cb_ref[...]
    prod = lax.dot_general(m, c, (((1,), (1,)), ((), ())),
                           preferred_element_type=jnp.float32)
    d = (jnp.sum(m * m, axis=1, keepdims=True) - 2.0 * prod
         + jnp.sum(c * c, axis=1)[None, :])
    dmin = jnp.min(d, axis=1, keepdims=True)
    iota = lax.broadcasted_iota(jnp.int32, d.shape, 1)
    idx = jnp.min(jnp.where(d == dmin, iota, K), axis=1, keepdims=True)
    codes_ref[...] = idx


MONO = -2**31


def _float_keys(s):
    u = lax.bitcast_convert_type(s, jnp.int32)
    m = u ^ (lax.shift_right_arithmetic(u, 31) | MONO)
    iota = lax.broadcasted_iota(jnp.int32, s.shape, 1)
    ka = (m & ~1023) | (1023 - iota)
    bits = ka ^ (~lax.shift_right_arithmetic(ka, 31) | MONO)
    return lax.bitcast_convert_type(bits, jnp.float32)


def _key_to_idx(mx):
    u = lax.bitcast_convert_type(mx, jnp.int32)
    m = u ^ (lax.shift_right_arithmetic(u, 31) | MONO)
    return 1023 - (m & 1023)


def _screen(P, L):
    keys = _float_keys(P + L)
    jis, sels = [], []
    for _ in range(NS):
        mx = jnp.max(keys, axis=1, keepdims=True)
        sel = keys == mx
        jis.append(_key_to_idx(mx))
        sels.append(sel)
        keys = jnp.where(sel, NEG, keys)
    ones = jnp.ones((K, 1), jnp.float32)
    dn = (((1,), (0,)), ((), ()))
    pvs, lvs = [], []
    for sel in sels:
        pvs.append(lax.dot_general(jnp.where(sel, P, 0.0), ones, dn,
                                   preferred_element_type=jnp.float32))
        lvs.append(lax.dot_general(jnp.where(sel, L, 0.0), ones, dn,
                                   preferred_element_type=jnp.float32))
    return (jnp.concatenate(jis, 1), jnp.concatenate(pvs, 1),
            jnp.concatenate(lvs, 1))


def _merge4(vj, fidx):
    outv, outf = [], []
    for _ in range(4):
        mx = jnp.max(vj, keepdims=True)
        f = jnp.min(jnp.where(vj == mx, fidx, float(2**23)), keepdims=True)
        vj = jnp.where(fidx == f, NEG, vj)
        outv.append(mx)
        outf.append(f)
    return jnp.concatenate(outv, 0), outf


def _combos(p1v, p2v, l1v, l2v, ai, ci, scv):
    vs, eis = [], []
    for r in range(NS):
        p1c = p1v[:, r:r + 1]
        l1c = l1v[:, r:r + 1]
        if scv is None:
            v = ((p1c + p2v) + l1c) + l2v
        else:
            v = (((scv + p1c) + p2v) + l1c) + l2v
        vs.append(v)
        eis.append(ai[:, r:r + 1] * K + ci)
    return jnp.concatenate(vs, 1), jnp.concatenate(eis, 1)


def _bs_body(codes_ref, p1_ref, p2_ref, l1_ref, l2_ref, seq_ref,
             rp_ref, rl_ref, h1_ref, h2_ref, bp_ref, z10_ref, z20_ref):
    m0 = codes_ref[0, 0]

    Pi = jnp.concatenate([p1_ref[pl.ds(0, 1), :], p2_ref[pl.ds(0, 1), :]], 0)
    Li = jnp.concatenate([l1_ref[pl.ds(m0, 1), :], l2_ref[pl.ds(m0, 1), :]], 0)
    ji, pv, lv = _screen(Pi, Li)
    v, ei = _combos(pv[0:1], pv[1:2], lv[0:1], lv[1:2],
                    ji[0:1], ji[1:2], None)
    scv0, f_l = _merge4(v, ei.astype(jnp.float32))
    zs = []
    for k in range(4):
        fk = f_l[k][0, 0].astype(jnp.int32)
        z1k = lax.shift_right_logical(fk, 10)
        z2k = jnp.bitwise_and(fk, K - 1)
        z10_ref[k] = z1k
        z20_ref[k] = z2k
        zs += [z1k, z2k]
    carry = tuple(zs) + (scv0,)

    def step(t, carry):
        (z10, z20, z11, z21, z12, z22, z13, z23, scv) = carry
        m_t = codes_ref[t, 0]
        rp_ref[pl.ds(0, 1), :] = p1_ref[pl.ds(z10, 1), :]
        rp_ref[pl.ds(1, 1), :] = p1_ref[pl.ds(z11, 1), :]
        rp_ref[pl.ds(2, 1), :] = p1_ref[pl.ds(z12, 1), :]
        rp_ref[pl.ds(3, 1), :] = p1_ref[pl.ds(z13, 1), :]
        rp_ref[pl.ds(4, 1), :] = p2_ref[pl.ds(z20, 1), :]
        rp_ref[pl.ds(5, 1), :] = p2_ref[pl.ds(z21, 1), :]
        rp_ref[pl.ds(6, 1), :] = p2_ref[pl.ds(z22, 1), :]
        rp_ref[pl.ds(7, 1), :] = p2_ref[pl.ds(z23, 1), :]
        l1r = l1_ref[pl.ds(m_t, 1), :]
        l2r = l2_ref[pl.ds(m_t, 1), :]
        rl_ref[pl.ds(0, 1), :] = l1r
        rl_ref[pl.ds(1, 1), :] = l1r
        rl_ref[pl.ds(2, 1), :] = l1r
        rl_ref[pl.ds(3, 1), :] = l1r
        rl_ref[pl.ds(4, 1), :] = l2r
        rl_ref[pl.ds(5, 1), :] = l2r
        rl_ref[pl.ds(6, 1), :] = l2r
        rl_ref[pl.ds(7, 1), :] = l2r
        P = rp_ref[...]
        L = rl_ref[...]
        ji, pv, lv = _screen(P, L)
        v, ei = _combos(pv[0:B], pv[B:2 * B], lv[0:B], lv[B:2 * B],
                        ji[0:B], ji[B:2 * B], scv)
        bio2 = lax.broadcasted_iota(jnp.int32, (B, NS * NS), 0)
        fi = bio2 * KK + ei
        scv_n, f_l = _merge4(v, fi.astype(jnp.float32))
        nzs = []
        for k in range(4):
            fk = f_l[k][0, 0].astype(jnp.int32)
            bk = lax.shift_right_logical(fk, 20)
            nz1 = jnp.bitwise_and(lax.shift_right_logical(fk, 10), K - 1)
            nz2 = jnp.bitwise_and(fk, K - 1)
            h1_ref[t - 1, k] = nz1
            h2_ref[t - 1, k] = nz2
            bp_ref[t - 1, k] = bk
            nzs += [nz1, nz2]
        return tuple(nzs) + (scv_n,)

    lax.fori_loop(1, T, step, tuple(carry))

    def back(j, b):
        i = T - 2 - j
        seq_ref[0, i + 1] = h1_ref[i, b]
        seq_ref[1, i + 1] = h2_ref[i, b]
        return bp_ref[i, b]

    b0 = lax.fori_loop(0, T - 1, back, jnp.int32(0))
    seq_ref[0, 0] = z10_ref[b0]
    seq_ref[1, 0] = z20_ref[b0]


def _decode_sc(seq_flat, codebook_pad):
    bpw = (2 * T) // NW

    @functools.partial(
        pl.kernel,
        out_type=jax.ShapeDtypeStruct((2 * T, 2 * D), jnp.float32),
        mesh=plsc.VectorSubcoreMesh(core_axis_name="c", subcore_axis_name="s"),
        scratch_types=[
            pltpu.VMEM((bpw,), jnp.int32),
            pltpu.VMEM((bpw, 2 * D), jnp.float32),
            pltpu.SemaphoreType.DMA,
        ],
    )
    def dec(seq_hbm, cb_hbm, out_hbm, idx_v, rows_v, sem):
        wid = lax.axis_index("s") * NC_SC + lax.axis_index("c")
        base = wid * bpw
        pltpu.sync_copy(seq_hbm.at[pl.ds(base, bpw)], idx_v)
        pltpu.async_copy(cb_hbm.at[idx_v], rows_v, sem).wait()
        pltpu.sync_copy(rows_v, out_hbm.at[pl.ds(base, bpw)])

    return dec(seq_flat, codebook_pad)


@jax.jit
def kernel(mixture, codebook, prior1, prior2, L1, L2):
    codes = pl.pallas_call(
        _encode_body,
        out_shape=jax.ShapeDtypeStruct((T, 1), jnp.int32),
    )(mixture, codebook)

    seq = pl.pallas_call(
        _bs_body,
        in_specs=[
            pl.BlockSpec(memory_space=pltpu.SMEM),
            pl.BlockSpec(memory_space=pltpu.VMEM),
            pl.BlockSpec(memory_space=pltpu.VMEM),
            pl.BlockSpec(memory_space=pltpu.VMEM),
            pl.BlockSpec(memory_space=pltpu.VMEM),
        ],
        out_shape=jax.ShapeDtypeStruct((2, T), jnp.int32),
        out_specs=pl.BlockSpec(memory_space=pltpu.SMEM),
        scratch_shapes=[
            pltpu.VMEM((2 * B, K), jnp.float32),
            pltpu.VMEM((2 * B, K), jnp.float32),
            pltpu.SMEM((T - 1, B), jnp.int32),
            pltpu.SMEM((T - 1, B), jnp.int32),
            pltpu.SMEM((T - 1, B), jnp.int32),
            pltpu.SMEM((B,), jnp.int32),
            pltpu.SMEM((B,), jnp.int32),
        ],
    )(codes, prior1, prior2, L1, L2)
    cb_pad = jnp.pad(codebook, ((0, 0), (0, D)))
    dec = _decode_sc(seq.reshape(2 * T), cb_pad)
    return dec[:, :D].reshape(2, T, D)


# speedup vs baseline: 2319.2225x; 1.0530x over previous
import jax
import jax.numpy as jnp
from jax import lax
from jax.experimental import pallas as pl
from jax.experimental.pallas import tpu as pltpu
from jax.experimental.pallas import tpu_sc as plsc

import functools

NC_SC = 2
NSUB = 16
NW = NC_SC * NSUB

K = 1024
D = 64
T = 256
B = 4
NS = 8
NEG = float("-inf")
BIGI = 2**31 - 1
KK = K * K
IMIN = -2**31


def _encode_body(mix_ref, cb_ref, codes_ref):
    m = mix_ref[...]
    c = cb_ref[...]
    prod = lax.dot_general(m, c, (((1,), (1,)), ((), ())),
                           preferred_element_type=jnp.float32)
    d = (jnp.sum(m * m, axis=1, keepdims=True) - 2.0 * prod
         + jnp.sum(c * c, axis=1)[None, :])
    dmin = jnp.min(d, axis=1, keepdims=True)
    iota = lax.broadcasted_iota(jnp.int32, d.shape, 1)
    idx = jnp.min(jnp.where(d == dmin, iota, K), axis=1, keepdims=True)
    codes_ref[...] = idx


MONO = -2**31


def _float_keys(s):
    u = lax.bitcast_convert_type(s, jnp.int32)
    m = u ^ (lax.shift_right_arithmetic(u, 31) | MONO)
    iota = lax.broadcasted_iota(jnp.int32, s.shape, 1)
    ka = (m & ~1023) | (1023 - iota)
    bits = ka ^ (~lax.shift_right_arithmetic(ka, 31) | MONO)
    return lax.bitcast_convert_type(bits, jnp.float32)


def _key_to_idx(mx):
    u = lax.bitcast_convert_type(mx, jnp.int32)
    m = u ^ (lax.shift_right_arithmetic(u, 31) | MONO)
    return 1023 - (m & 1023)


def _screen(P, L):
    keys = _float_keys(P + L)
    jis, sels = [], []
    for _ in range(NS):
        mx = jnp.max(keys, axis=1, keepdims=True)
        sel = keys == mx
        jis.append(_key_to_idx(mx))
        sels.append(sel)
        keys = jnp.where(sel, NEG, keys)
    pvs, lvs = [], []
    for sel in sels:
        pvs.append(jnp.sum(jnp.where(sel, P, 0.0), axis=1, keepdims=True))
        lvs.append(jnp.sum(jnp.where(sel, L, 0.0), axis=1, keepdims=True))
    return (jnp.concatenate(jis, 1), jnp.concatenate(pvs, 1),
            jnp.concatenate(lvs, 1))


def _merge4(vj, fidx):
    outv, outf = [], []
    for _ in range(4):
        mx = jnp.max(vj, keepdims=True)
        f = jnp.min(jnp.where(vj == mx, fidx, float(2**23)), keepdims=True)
        vj = jnp.where(fidx == f, NEG, vj)
        outv.append(mx)
        outf.append(f)
    return jnp.concatenate(outv, 0), outf


def _combos(p1v, p2v, l1v, l2v, ai, ci, scv):
    vs, eis = [], []
    for r in range(NS):
        p1c = p1v[:, r:r + 1]
        l1c = l1v[:, r:r + 1]
        if scv is None:
            v = ((p1c + p2v) + l1c) + l2v
        else:
            v = (((scv + p1c) + p2v) + l1c) + l2v
        vs.append(v)
        eis.append(ai[:, r:r + 1] * K + ci)
    return jnp.concatenate(vs, 1), jnp.concatenate(eis, 1)


def _bs_body(codes_ref, p1_ref, p2_ref, l1_ref, l2_ref, seq_ref,
             rp_ref, rl_ref, h1_ref, h2_ref, bp_ref, z10_ref, z20_ref):
    m0 = codes_ref[0, 0]

    Pi = jnp.concatenate([p1_ref[pl.ds(0, 1), :], p2_ref[pl.ds(0, 1), :]], 0)
    Li = jnp.concatenate([l1_ref[pl.ds(m0, 1), :], l2_ref[pl.ds(m0, 1), :]], 0)
    ji, pv, lv = _screen(Pi, Li)
    v, ei = _combos(pv[0:1], pv[1:2], lv[0:1], lv[1:2],
                    ji[0:1], ji[1:2], None)
    scv0, f_l = _merge4(v, ei.astype(jnp.float32))
    zs = []
    for k in range(4):
        fk = f_l[k][0, 0].astype(jnp.int32)
        z1k = lax.shift_right_logical(fk, 10)
        z2k = jnp.bitwise_and(fk, K - 1)
        z10_ref[k] = z1k
        z20_ref[k] = z2k
        zs += [z1k, z2k]
    carry = tuple(zs) + (scv0,)

    def step(t, carry):
        (z10, z20, z11, z21, z12, z22, z13, z23, scv) = carry
        m_t = codes_ref[t, 0]
        rp_ref[pl.ds(0, 1), :] = p1_ref[pl.ds(z10, 1), :]
        rp_ref[pl.ds(1, 1), :] = p1_ref[pl.ds(z11, 1), :]
        rp_ref[pl.ds(2, 1), :] = p1_ref[pl.ds(z12, 1), :]
        rp_ref[pl.ds(3, 1), :] = p1_ref[pl.ds(z13, 1), :]
        rp_ref[pl.ds(4, 1), :] = p2_ref[pl.ds(z20, 1), :]
        rp_ref[pl.ds(5, 1), :] = p2_ref[pl.ds(z21, 1), :]
        rp_ref[pl.ds(6, 1), :] = p2_ref[pl.ds(z22, 1), :]
        rp_ref[pl.ds(7, 1), :] = p2_ref[pl.ds(z23, 1), :]
        l1r = l1_ref[pl.ds(m_t, 1), :]
        l2r = l2_ref[pl.ds(m_t, 1), :]
        rl_ref[pl.ds(0, 1), :] = l1r
        rl_ref[pl.ds(1, 1), :] = l1r
        rl_ref[pl.ds(2, 1), :] = l1r
        rl_ref[pl.ds(3, 1), :] = l1r
        rl_ref[pl.ds(4, 1), :] = l2r
        rl_ref[pl.ds(5, 1), :] = l2r
        rl_ref[pl.ds(6, 1), :] = l2r
        rl_ref[pl.ds(7, 1), :] = l2r
        P = rp_ref[...]
        L = rl_ref[...]
        ji, pv, lv = _screen(P, L)
        v, ei = _combos(pv[0:B], pv[B:2 * B], lv[0:B], lv[B:2 * B],
                        ji[0:B], ji[B:2 * B], scv)
        bio2 = lax.broadcasted_iota(jnp.int32, (B, NS * NS), 0)
        fi = bio2 * KK + ei
        scv_n, f_l = _merge4(v, fi.astype(jnp.float32))
        nzs = []
        for k in range(4):
            fk = f_l[k][0, 0].astype(jnp.int32)
            bk = lax.shift_right_logical(fk, 20)
            nz1 = jnp.bitwise_and(lax.shift_right_logical(fk, 10), K - 1)
            nz2 = jnp.bitwise_and(fk, K - 1)
            h1_ref[t - 1, k] = nz1
            h2_ref[t - 1, k] = nz2
            bp_ref[t - 1, k] = bk
            nzs += [nz1, nz2]
        return tuple(nzs) + (scv_n,)

    lax.fori_loop(1, T, step, tuple(carry))

    def back(j, b):
        i = T - 2 - j
        seq_ref[0, i + 1] = h1_ref[i, b]
        seq_ref[1, i + 1] = h2_ref[i, b]
        return bp_ref[i, b]

    b0 = lax.fori_loop(0, T - 1, back, jnp.int32(0))
    seq_ref[0, 0] = z10_ref[b0]
    seq_ref[1, 0] = z20_ref[b0]


def _decode_sc(seq_flat, codebook_pad):
    bpw = (2 * T) // NW

    @functools.partial(
        pl.kernel,
        out_type=jax.ShapeDtypeStruct((2 * T, 2 * D), jnp.float32),
        mesh=plsc.VectorSubcoreMesh(core_axis_name="c", subcore_axis_name="s"),
        scratch_types=[
            pltpu.VMEM((bpw,), jnp.int32),
            pltpu.VMEM((bpw, 2 * D), jnp.float32),
            pltpu.SemaphoreType.DMA,
        ],
    )
    def dec(seq_hbm, cb_hbm, out_hbm, idx_v, rows_v, sem):
        wid = lax.axis_index("s") * NC_SC + lax.axis_index("c")
        base = wid * bpw
        pltpu.sync_copy(seq_hbm.at[pl.ds(base, bpw)], idx_v)
        pltpu.async_copy(cb_hbm.at[idx_v], rows_v, sem).wait()
        pltpu.sync_copy(rows_v, out_hbm.at[pl.ds(base, bpw)])

    return dec(seq_flat, codebook_pad)


@jax.jit
def kernel(mixture, codebook, prior1, prior2, L1, L2):
    codes = pl.pallas_call(
        _encode_body,
        out_shape=jax.ShapeDtypeStruct((T, 1), jnp.int32),
    )(mixture, codebook)

    seq = pl.pallas_call(
        _bs_body,
        in_specs=[
            pl.BlockSpec(memory_space=pltpu.SMEM),
            pl.BlockSpec(memory_space=pltpu.VMEM),
            pl.BlockSpec(memory_space=pltpu.VMEM),
            pl.BlockSpec(memory_space=pltpu.VMEM),
            pl.BlockSpec(memory_space=pltpu.VMEM),
        ],
        out_shape=jax.ShapeDtypeStruct((2, T), jnp.int32),
        out_specs=pl.BlockSpec(memory_space=pltpu.SMEM),
        scratch_shapes=[
            pltpu.VMEM((2 * B, K), jnp.float32),
            pltpu.VMEM((2 * B, K), jnp.float32),
            pltpu.SMEM((T - 1, B), jnp.int32),
            pltpu.SMEM((T - 1, B), jnp.int32),
            pltpu.SMEM((T - 1, B), jnp.int32),
            pltpu.SMEM((B,), jnp.int32),
            pltpu.SMEM((B,), jnp.int32),
        ],
    )(codes, prior1, prior2, L1, L2)
    cb_pad = jnp.pad(codebook, ((0, 0), (0, D)))
    dec = _decode_sc(seq.reshape(2 * T), cb_pad)
    return dec[:, :D].reshape(2, T, D)


# speedup vs baseline: 2513.5249x; 1.0838x over previous
import jax
import jax.numpy as jnp
from jax import lax
from jax.experimental import pallas as pl
from jax.experimental.pallas import tpu as pltpu
from jax.experimental.pallas import tpu_sc as plsc

import functools

NC_SC = 2
NSUB = 16
NW = NC_SC * NSUB

K = 1024
D = 64
T = 256
B = 4
NS = 6
NEG = float("-inf")
BIGI = 2**31 - 1
KK = K * K
IMIN = -2**31


def _encode_body(mix_ref, cb_ref, codes_ref):
    m = mix_ref[...]
    c = cb_ref[...]
    prod = lax.dot_general(m, c, (((1,), (1,)), ((), ())),
                           preferred_element_type=jnp.float32)
    d = (jnp.sum(m * m, axis=1, keepdims=True) - 2.0 * prod
         + jnp.sum(c * c, axis=1)[None, :])
    dmin = jnp.min(d, axis=1, keepdims=True)
    iota = lax.broadcasted_iota(jnp.int32, d.shape, 1)
    idx = jnp.min(jnp.where(d == dmin, iota, K), axis=1, keepdims=True)
    codes_ref[...] = idx


MONO = -2**31


def _float_keys(s):
    u = lax.bitcast_convert_type(s, jnp.int32)
    m = u ^ (lax.shift_right_arithmetic(u, 31) | MONO)
    iota = lax.broadcasted_iota(jnp.int32, s.shape, 1)
    ka = (m & ~1023) | (1023 - iota)
    bits = ka ^ (~lax.shift_right_arithmetic(ka, 31) | MONO)
    return lax.bitcast_convert_type(bits, jnp.float32)


def _key_to_idx(mx):
    u = lax.bitcast_convert_type(mx, jnp.int32)
    m = u ^ (lax.shift_right_arithmetic(u, 31) | MONO)
    return 1023 - (m & 1023)


def _screen(P, L):
    keys = _float_keys(P + L)
    jis, sels = [], []
    for _ in range(NS):
        mx = jnp.max(keys, axis=1, keepdims=True)
        sel = keys == mx
        jis.append(_key_to_idx(mx))
        sels.append(sel)
        keys = jnp.where(sel, NEG, keys)
    pvs, lvs = [], []
    for sel in sels:
        pvs.append(jnp.sum(jnp.where(sel, P, 0.0), axis=1, keepdims=True))
        lvs.append(jnp.sum(jnp.where(sel, L, 0.0), axis=1, keepdims=True))
    return (jnp.concatenate(jis, 1), jnp.concatenate(pvs, 1),
            jnp.concatenate(lvs, 1))


def _merge4(vj, fidx):
    outv, outf = [], []
    for _ in range(4):
        mx = jnp.max(vj, keepdims=True)
        f = jnp.min(jnp.where(vj == mx, fidx, float(2**23)), keepdims=True)
        vj = jnp.where(fidx == f, NEG, vj)
        outv.append(mx)
        outf.append(f)
    return jnp.concatenate(outv, 0), outf


def _combos(p1v, p2v, l1v, l2v, ai, ci, scv):
    vs, eis = [], []
    for r in range(NS):
        p1c = p1v[:, r:r + 1]
        l1c = l1v[:, r:r + 1]
        if scv is None:
            v = ((p1c + p2v) + l1c) + l2v
        else:
            v = (((scv + p1c) + p2v) + l1c) + l2v
        vs.append(v)
        eis.append(ai[:, r:r + 1] * K + ci)
    return jnp.concatenate(vs, 1), jnp.concatenate(eis, 1)


def _bs_body(codes_ref, p1_ref, p2_ref, l1_ref, l2_ref, seq_ref,
             rp_ref, rl_ref, h1_ref, h2_ref, bp_ref, z10_ref, z20_ref):
    m0 = codes_ref[0, 0]

    Pi = jnp.concatenate([p1_ref[pl.ds(0, 1), :], p2_ref[pl.ds(0, 1), :]], 0)
    Li = jnp.concatenate([l1_ref[pl.ds(m0, 1), :], l2_ref[pl.ds(m0, 1), :]], 0)
    ji, pv, lv = _screen(Pi, Li)
    v, ei = _combos(pv[0:1], pv[1:2], lv[0:1], lv[1:2],
                    ji[0:1], ji[1:2], None)
    scv0, f_l = _merge4(v, ei.astype(jnp.float32))
    zs = []
    for k in range(4):
        fk = f_l[k][0, 0].astype(jnp.int32)
        z1k = lax.shift_right_logical(fk, 10)
        z2k = jnp.bitwise_and(fk, K - 1)
        z10_ref[k] = z1k
        z20_ref[k] = z2k
        zs += [z1k, z2k]
    carry = tuple(zs) + (scv0,)

    def step(t, carry):
        (z10, z20, z11, z21, z12, z22, z13, z23, scv) = carry
        m_t = codes_ref[t, 0]
        rp_ref[pl.ds(0, 1), :] = p1_ref[pl.ds(z10, 1), :]
        rp_ref[pl.ds(1, 1), :] = p1_ref[pl.ds(z11, 1), :]
        rp_ref[pl.ds(2, 1), :] = p1_ref[pl.ds(z12, 1), :]
        rp_ref[pl.ds(3, 1), :] = p1_ref[pl.ds(z13, 1), :]
        rp_ref[pl.ds(4, 1), :] = p2_ref[pl.ds(z20, 1), :]
        rp_ref[pl.ds(5, 1), :] = p2_ref[pl.ds(z21, 1), :]
        rp_ref[pl.ds(6, 1), :] = p2_ref[pl.ds(z22, 1), :]
        rp_ref[pl.ds(7, 1), :] = p2_ref[pl.ds(z23, 1), :]
        l1r = l1_ref[pl.ds(m_t, 1), :]
        l2r = l2_ref[pl.ds(m_t, 1), :]
        rl_ref[pl.ds(0, 1), :] = l1r
        rl_ref[pl.ds(1, 1), :] = l1r
        rl_ref[pl.ds(2, 1), :] = l1r
        rl_ref[pl.ds(3, 1), :] = l1r
        rl_ref[pl.ds(4, 1), :] = l2r
        rl_ref[pl.ds(5, 1), :] = l2r
        rl_ref[pl.ds(6, 1), :] = l2r
        rl_ref[pl.ds(7, 1), :] = l2r
        P = rp_ref[...]
        L = rl_ref[...]
        ji, pv, lv = _screen(P, L)
        v, ei = _combos(pv[0:B], pv[B:2 * B], lv[0:B], lv[B:2 * B],
                        ji[0:B], ji[B:2 * B], scv)
        bio2 = lax.broadcasted_iota(jnp.int32, (B, NS * NS), 0)
        fi = bio2 * KK + ei
        scv_n, f_l = _merge4(v, fi.astype(jnp.float32))
        nzs = []
        for k in range(4):
            fk = f_l[k][0, 0].astype(jnp.int32)
            bk = lax.shift_right_logical(fk, 20)
            nz1 = jnp.bitwise_and(lax.shift_right_logical(fk, 10), K - 1)
            nz2 = jnp.bitwise_and(fk, K - 1)
            h1_ref[t - 1, k] = nz1
            h2_ref[t - 1, k] = nz2
            bp_ref[t - 1, k] = bk
            nzs += [nz1, nz2]
        return tuple(nzs) + (scv_n,)

    lax.fori_loop(1, T, step, tuple(carry))

    def back(j, b):
        i = T - 2 - j
        seq_ref[0, i + 1] = h1_ref[i, b]
        seq_ref[1, i + 1] = h2_ref[i, b]
        return bp_ref[i, b]

    b0 = lax.fori_loop(0, T - 1, back, jnp.int32(0))
    seq_ref[0, 0] = z10_ref[b0]
    seq_ref[1, 0] = z20_ref[b0]


def _decode_sc(seq_flat, codebook_pad):
    bpw = (2 * T) // NW

    @functools.partial(
        pl.kernel,
        out_type=jax.ShapeDtypeStruct((2 * T, 2 * D), jnp.float32),
        mesh=plsc.VectorSubcoreMesh(core_axis_name="c", subcore_axis_name="s"),
        scratch_types=[
            pltpu.VMEM((bpw,), jnp.int32),
            pltpu.VMEM((bpw, 2 * D), jnp.float32),
            pltpu.SemaphoreType.DMA,
        ],
    )
    def dec(seq_hbm, cb_hbm, out_hbm, idx_v, rows_v, sem):
        wid = lax.axis_index("s") * NC_SC + lax.axis_index("c")
        base = wid * bpw
        pltpu.sync_copy(seq_hbm.at[pl.ds(base, bpw)], idx_v)
        pltpu.async_copy(cb_hbm.at[idx_v], rows_v, sem).wait()
        pltpu.sync_copy(rows_v, out_hbm.at[pl.ds(base, bpw)])

    return dec(seq_flat, codebook_pad)


@jax.jit
def kernel(mixture, codebook, prior1, prior2, L1, L2):
    codes = pl.pallas_call(
        _encode_body,
        out_shape=jax.ShapeDtypeStruct((T, 1), jnp.int32),
    )(mixture, codebook)

    seq = pl.pallas_call(
        _bs_body,
        in_specs=[
            pl.BlockSpec(memory_space=pltpu.SMEM),
            pl.BlockSpec(memory_space=pltpu.VMEM),
            pl.BlockSpec(memory_space=pltpu.VMEM),
            pl.BlockSpec(memory_space=pltpu.VMEM),
            pl.BlockSpec(memory_space=pltpu.VMEM),
        ],
        out_shape=jax.ShapeDtypeStruct((2, T), jnp.int32),
        out_specs=pl.BlockSpec(memory_space=pltpu.SMEM),
        scratch_shapes=[
            pltpu.VMEM((2 * B, K), jnp.float32),
            pltpu.VMEM((2 * B, K), jnp.float32),
            pltpu.SMEM((T - 1, B), jnp.int32),
            pltpu.SMEM((T - 1, B), jnp.int32),
            pltpu.SMEM((T - 1, B), jnp.int32),
            pltpu.SMEM((B,), jnp.int32),
            pltpu.SMEM((B,), jnp.int32),
        ],
    )(codes, prior1, prior2, L1, L2)
    cb_pad = jnp.pad(codebook, ((0, 0), (0, D)))
    dec = _decode_sc(seq.reshape(2 * T), cb_pad)
    return dec[:, :D].reshape(2, T, D)
